# Initial kernel scaffold; baseline (speedup 1.0000x reference)
import jax
import jax.numpy as jnp
from jax.experimental import pallas as pl


def kernel(for_gen, x, edge_index, graph_ids, W1, b1, W2, b2, Wc, bc):
    raise NotImplementedError("write your pallas kernel here")


# speedup vs baseline: 6.3757x; 6.3757x over previous
import functools

import jax
import jax.numpy as jnp
from jax import lax
from jax.experimental import pallas as pl
from jax.experimental.pallas import tpu as pltpu
from jax.experimental.pallas import tpu_sc as plsc

N = 100000
E = 6400000
G = 64
IN_DIM = 3
H = 64

NC, NS, NW = 2, 16, 32
NPAD = 100352
BN1 = 3584
BN2 = 2048
Q = 200704
EPAD = NW * Q
CHUNK = 2048
NCHUNK = Q // CHUNK
SUBK = 128
NSUB = CHUNK // SUBK
DEG_CHUNK = 4096
DEG_NCHUNK = (2 * Q) // DEG_CHUNK

_f32 = jnp.float32
_i32 = jnp.int32

_MESH = plsc.VectorSubcoreMesh(core_axis_name="c", subcore_axis_name="s")
_SC_PARAMS = pltpu.CompilerParams(
    needs_layout_passes=False, use_tc_tiling_on_sc=False
)


def _deg_body(ei, out, hist, ibuf, sem):
    del sem
    c = lax.axis_index("c")
    s = lax.axis_index("s")
    zv = jnp.zeros((16,), _f32)

    def zero(i, carry):
        hist[pl.ds(i * 16, 16)] = zv
        return carry

    lax.fori_loop(0, NPAD // 16, zero, 0)

    ones = jnp.ones((16,), _f32)
    base = c * EPAD + s * (2 * Q)

    def chunk(j, carry):
        pltpu.sync_copy(ei.at[pl.ds(base + j * DEG_CHUNK, DEG_CHUNK)], ibuf)

        def inner(i, carry2):
            idx = ibuf[pl.ds(i * 16, 16)]
            plsc.addupdate_scatter(hist, [idx], ones)
            return carry2

        lax.fori_loop(0, DEG_CHUNK // 16, inner, 0)
        return carry

    lax.fori_loop(0, DEG_NCHUNK, chunk, 0)
    pltpu.sync_copy(hist, out.at[c, s])


_deg_call = functools.partial(
    pl.kernel,
    out_type=jax.ShapeDtypeStruct((NC, NS, NPAD), _f32),
    mesh=_MESH,
    compiler_params=_SC_PARAMS,
    scratch_types=[
        pltpu.VMEM((NPAD,), _f32),
        pltpu.VMEM((DEG_CHUNK,), _i32),
        pltpu.SemaphoreType.DMA,
    ],
)(_deg_body)


def _make_seg_body(ng):
    def body(ei, table, z16, out, acc, sbuf, dbufs, rows, sem):
        c = lax.axis_index("c")
        s = lax.axis_index("s")
        t = c * NS + s
        base = t * Q
        rows_per_tile = NPAD // NS

        for q in range(ng):
            pltpu.sync_copy(
                z16.at[pl.ds(s * rows_per_tile, rows_per_tile)],
                acc.at[pl.ds(s * rows_per_tile, rows_per_tile)],
            )
            plsc.subcore_barrier()

            def chunk(j, carry):
                off = base + j * CHUNK
                pltpu.sync_copy(ei.at[pl.ds(off, CHUNK)], sbuf)
                for k in range(NSUB):
                    pltpu.sync_copy(
                        ei.at[pl.ds(EPAD + off + k * SUBK, SUBK)], dbufs[k]
                    )
                for k in range(NSUB):
                    pltpu.async_copy(
                        table.at[q].at[sbuf.at[pl.ds(k * SUBK, SUBK)]],
                        rows,
                        sem,
                    ).wait()
                    pltpu.sync_copy(rows, acc.at[dbufs[k]], add=True)
                return carry

            lax.fori_loop(0, NCHUNK, chunk, 0)
            plsc.subcore_barrier()
            pltpu.sync_copy(
                acc.at[pl.ds(s * rows_per_tile, rows_per_tile)],
                out.at[c, q, pl.ds(s * rows_per_tile, rows_per_tile)],
            )
            plsc.subcore_barrier()

    return body


def _make_seg_call(ng):
    return functools.partial(
        pl.kernel,
        out_type=jax.ShapeDtypeStruct((NC, ng, NPAD, 16), _f32),
        mesh=_MESH,
        compiler_params=_SC_PARAMS,
        scratch_types=[
            pltpu.VMEM_SHARED((NPAD, 16), _f32),
            pltpu.VMEM((CHUNK,), _i32),
            [pltpu.VMEM((SUBK,), _i32) for _ in range(NSUB)],
            pltpu.VMEM((SUBK, 16), _f32),
            pltpu.SemaphoreType.DMA,
        ],
    )(_make_seg_body(ng))


_l1_call = _make_seg_call(1)
_l2_call = _make_seg_call(4)


def _tc1_body(degp, xp, x16, a_out, b_out):
    d = jnp.sum(degp[...], axis=1)
    a = lax.rsqrt(jnp.maximum(d[0], 1.0))
    b = lax.rsqrt(jnp.maximum(d[1], 1.0))
    a_out[...] = a[:, None]
    b_out[...] = b[:, None]
    xb = xp[...] * a[:, None]
    x16[...] = jnp.concatenate([xb, jnp.zeros_like(xb)], axis=1)


def _tc1(degp, xp):
    return pl.pallas_call(
        _tc1_body,
        grid=(NPAD // BN1,),
        in_specs=[
            pl.BlockSpec((NC, NS, BN1), lambda i: (0, 0, i)),
            pl.BlockSpec((BN1, 8), lambda i: (i, 0)),
        ],
        out_specs=[
            pl.BlockSpec((BN1, 16), lambda i: (i, 0)),
            pl.BlockSpec((BN1, 1), lambda i: (i, 0)),
            pl.BlockSpec((BN1, 1), lambda i: (i, 0)),
        ],
        out_shape=[
            jax.ShapeDtypeStruct((NPAD, 16), _f32),
            jax.ShapeDtypeStruct((NPAD, 1), _f32),
            jax.ShapeDtypeStruct((NPAD, 1), _f32),
        ],
    )(degp, xp)


def _tc2_body(s1p, b_in, a_in, w1, b1, msg):
    s1 = s1p[0, 0] + s1p[1, 0]
    z = (
        jnp.dot(s1 * b_in[...], w1[...], preferred_element_type=_f32)
        + b1[...]
    )
    m = jnp.maximum(z, 0.0) * a_in[...]
    for q in range(4):
        msg[q] = m[:, q * 16 : (q + 1) * 16]


def _tc2(s1p, b_arr, a_arr, w1p, b1r):
    return pl.pallas_call(
        _tc2_body,
        grid=(NPAD // BN2,),
        in_specs=[
            pl.BlockSpec((NC, 1, BN2, 16), lambda i: (0, 0, i, 0)),
            pl.BlockSpec((BN2, 1), lambda i: (i, 0)),
            pl.BlockSpec((BN2, 1), lambda i: (i, 0)),
            pl.BlockSpec((16, H), lambda i: (0, 0)),
            pl.BlockSpec((1, H), lambda i: (0, 0)),
        ],
        out_specs=pl.BlockSpec((4, BN2, 16), lambda i: (0, i, 0)),
        out_shape=jax.ShapeDtypeStruct((4, NPAD, 16), _f32),
    )(s1p, b_arr, a_arr, w1p, b1r)


def _tc3_body(fg, s2p, b_in, gid, w2, b2, wc, bc, loss, hsum, cnt):
    i = pl.program_id(0)

    @pl.when(i == 0)
    def _():
        hsum[...] = jnp.zeros_like(hsum)
        cnt[...] = jnp.zeros_like(cnt)

    sp = s2p[0] + s2p[1]
    s2 = jnp.concatenate([sp[q] for q in range(4)], axis=1)
    z = (
        jnp.dot(s2 * b_in[...], w2[...], preferred_element_type=_f32)
        + b2[...]
    )
    h2 = jnp.maximum(z, 0.0)
    oh = (gid[...] == lax.broadcasted_iota(_i32, (1, G), 1)).astype(_f32)
    hsum[...] += lax.dot_general(
        oh, h2, (((0,), (0,)), ((), ())), preferred_element_type=_f32
    )
    cnt[...] += jnp.sum(oh, axis=0, keepdims=True)

    @pl.when(i == NPAD // BN2 - 1)
    def _():
        hg = hsum[...] / jnp.maximum(cnt[...], 1.0).T
        sc = jnp.dot(hg, wc[...], preferred_element_type=_f32) + bc[...]
        svec = sc[:, 0:1]
        gen = jnp.mean((svec - 1.0) ** 2) * 0.5
        disc = jnp.mean(svec**2) * 0.5
        loss[...] = jnp.where(fg[0, 0] != 0, gen, disc).reshape(1, 1)


def _tc3(fg, s2, b_arr, gid, w2, b2r, wcp, bcp):
    return pl.pallas_call(
        _tc3_body,
        grid=(NPAD // BN2,),
        in_specs=[
            pl.BlockSpec((1, 1), lambda i: (0, 0)),
            pl.BlockSpec((NC, 4, BN2, 16), lambda i: (0, 0, i, 0)),
            pl.BlockSpec((BN2, 1), lambda i: (i, 0)),
            pl.BlockSpec((BN2, 1), lambda i: (i, 0)),
            pl.BlockSpec((H, H), lambda i: (0, 0)),
            pl.BlockSpec((1, H), lambda i: (0, 0)),
            pl.BlockSpec((H, 8), lambda i: (0, 0)),
            pl.BlockSpec((1, 8), lambda i: (0, 0)),
        ],
        out_specs=pl.BlockSpec((1, 1), lambda i: (0, 0)),
        out_shape=jax.ShapeDtypeStruct((1, 1), _f32),
        scratch_shapes=[
            pltpu.VMEM((G, H), _f32),
            pltpu.VMEM((1, G), _f32),
        ],
    )(fg, s2, b_arr, gid, w2, b2r, wcp, bcp)


def kernel(for_gen, x, edge_index, graph_ids, W1, b1, W2, b2, Wc, bc):
    ei_pad = jnp.pad(
        edge_index, ((0, 0), (0, EPAD - E)), constant_values=NPAD - 1
    )
    ei_flat = ei_pad.reshape(2 * EPAD)

    degp = _deg_call(ei_flat)

    xp = jnp.zeros((NPAD, 8), _f32).at[:N, :IN_DIM].set(x)
    x16, a_arr, b_arr = _tc1(degp, xp)

    z16 = jnp.zeros((NPAD, 16), _f32)
    s1p = _l1_call(ei_flat, x16.reshape(1, NPAD, 16), z16)

    w1p = jnp.zeros((16, H), _f32).at[:IN_DIM].set(W1)
    msg = _tc2(s1p, b_arr, a_arr, w1p, b1.reshape(1, H))

    s2p = _l2_call(ei_flat, msg, z16)

    gid = jnp.pad(graph_ids, (0, NPAD - N), constant_values=G).reshape(
        NPAD, 1
    )
    wcp = jnp.zeros((H, 8), _f32).at[:, :1].set(Wc)
    bcp = jnp.zeros((1, 8), _f32).at[0, 0].set(bc[0])
    fg = jnp.asarray(for_gen, _i32).reshape(1, 1)
    loss = _tc3(fg, s2p, b_arr, gid, W2, b2.reshape(1, H), wcp, bcp)
    return loss[0, 0]


# speedup vs baseline: 9.8951x; 1.5520x over previous
import functools

import jax
import jax.numpy as jnp
from jax import lax
from jax.experimental import pallas as pl
from jax.experimental.pallas import tpu as pltpu
from jax.experimental.pallas import tpu_sc as plsc

N = 100000
E = 6400000
G = 64
IN_DIM = 3
H = 64

NC, NS, NW = 2, 16, 32
NPAD = 100352
BN1 = 3584
BN2 = 2048
Q = 200704
EPAD = NW * Q
CHUNK = 2048
NCHUNK = Q // CHUNK
SUBK = 128
NSUB = CHUNK // SUBK
DEG_CHUNK = 4096
DEG_NCHUNK = (2 * Q) // DEG_CHUNK

_f32 = jnp.float32
_i32 = jnp.int32

_MESH = plsc.VectorSubcoreMesh(core_axis_name="c", subcore_axis_name="s")
_SC_PARAMS = pltpu.CompilerParams(
    needs_layout_passes=False, use_tc_tiling_on_sc=False
)


def _deg_body(ei, out, hist, ibufs, sem):
    c = lax.axis_index("c")
    s = lax.axis_index("s")
    zv = jnp.zeros((16,), _f32)

    def zero(i, carry):
        hist[pl.ds(i * 16, 16)] = zv
        return carry

    lax.fori_loop(0, NPAD // 16, zero, 0)

    ones = jnp.ones((16,), _f32)
    base = c * EPAD + s * (2 * Q)
    nvec = DEG_CHUNK // 16

    def scan(buf):
        def inner(i, carry2):
            plsc.addupdate_scatter(hist, [buf[pl.ds(i * 16, 16)]], ones)
            return carry2

        lax.fori_loop(0, nvec, inner, 0)

    def fire(j, buf):
        jc = jnp.where(j < DEG_NCHUNK, j, 0)
        return pltpu.async_copy(
            ei.at[pl.ds(base + jc * DEG_CHUNK, DEG_CHUNK)], buf, sem
        )

    def drain(buf):
        pltpu.make_async_copy(ei.at[pl.ds(base, DEG_CHUNK)], buf, sem).wait()

    fire(0, ibufs[0])

    def chunk2(j2, carry):
        j0 = 2 * j2
        drain(ibufs[0])
        fire(j0 + 1, ibufs[1])
        scan(ibufs[0])
        drain(ibufs[1])
        fire(j0 + 2, ibufs[0])
        scan(ibufs[1])
        return carry

    lax.fori_loop(0, DEG_NCHUNK // 2, chunk2, 0)
    drain(ibufs[0])
    pltpu.sync_copy(hist, out.at[c, s])


_deg_call = functools.partial(
    pl.kernel,
    out_type=jax.ShapeDtypeStruct((NC, NS, NPAD), _f32),
    mesh=_MESH,
    compiler_params=_SC_PARAMS,
    scratch_types=[
        pltpu.VMEM((NPAD,), _f32),
        [pltpu.VMEM((DEG_CHUNK,), _i32) for _ in range(2)],
        pltpu.SemaphoreType.DMA,
    ],
)(_deg_body)


def _make_seg_body(ng):
    def body(ei, table, z16, out, acc, sbuf, dbufs, rows2, isem, gsem):
        c = lax.axis_index("c")
        s = lax.axis_index("s")
        t = c * NS + s
        base = t * Q
        rows_per_tile = NPAD // NS

        for q in range(ng):
            pltpu.sync_copy(
                z16.at[pl.ds(s * rows_per_tile, rows_per_tile)],
                acc.at[pl.ds(s * rows_per_tile, rows_per_tile)],
            )
            plsc.subcore_barrier()

            def chunk(j, carry):
                off = base + j * CHUNK
                stage = [pltpu.async_copy(ei.at[pl.ds(off, CHUNK)], sbuf, isem)]
                for k in range(NSUB):
                    stage.append(
                        pltpu.async_copy(
                            ei.at[pl.ds(EPAD + off + k * SUBK, SUBK)],
                            dbufs[k],
                            isem,
                        )
                    )
                for d in stage:
                    d.wait()
                g = pltpu.async_copy(
                    table.at[q].at[sbuf.at[pl.ds(0, SUBK)]], rows2[0], gsem[0]
                )
                for k in range(NSUB):
                    g.wait()
                    if k + 1 < NSUB:
                        g = pltpu.async_copy(
                            table.at[q].at[
                                sbuf.at[pl.ds((k + 1) * SUBK, SUBK)]
                            ],
                            rows2[(k + 1) % 2],
                            gsem[(k + 1) % 2],
                        )
                    pltpu.sync_copy(rows2[k % 2], acc.at[dbufs[k]], add=True)
                return carry

            lax.fori_loop(0, NCHUNK, chunk, 0)
            plsc.subcore_barrier()
            pltpu.sync_copy(
                acc.at[pl.ds(s * rows_per_tile, rows_per_tile)],
                out.at[c, q, pl.ds(s * rows_per_tile, rows_per_tile)],
            )
            plsc.subcore_barrier()

    return body


def _make_seg_call(ng):
    return functools.partial(
        pl.kernel,
        out_type=jax.ShapeDtypeStruct((NC, ng, NPAD, 16), _f32),
        mesh=_MESH,
        compiler_params=_SC_PARAMS,
        scratch_types=[
            pltpu.VMEM_SHARED((NPAD, 16), _f32),
            pltpu.VMEM((CHUNK,), _i32),
            [pltpu.VMEM((SUBK,), _i32) for _ in range(NSUB)],
            [pltpu.VMEM((SUBK, 16), _f32) for _ in range(2)],
            pltpu.SemaphoreType.DMA,
            [pltpu.SemaphoreType.DMA for _ in range(2)],
        ],
    )(_make_seg_body(ng))


_l1_call = _make_seg_call(1)
_l2_call = _make_seg_call(4)


def _tc1_body(degp, xp, x16, a_out, b_out):
    d = jnp.sum(degp[...], axis=1)
    a = lax.rsqrt(jnp.maximum(d[0], 1.0))
    b = lax.rsqrt(jnp.maximum(d[1], 1.0))
    a_out[...] = a[:, None]
    b_out[...] = b[:, None]
    xb = xp[...] * a[:, None]
    x16[...] = jnp.concatenate([xb, jnp.zeros_like(xb)], axis=1)


def _tc1(degp, xp):
    return pl.pallas_call(
        _tc1_body,
        grid=(NPAD // BN1,),
        in_specs=[
            pl.BlockSpec((NC, NS, BN1), lambda i: (0, 0, i)),
            pl.BlockSpec((BN1, 8), lambda i: (i, 0)),
        ],
        out_specs=[
            pl.BlockSpec((BN1, 16), lambda i: (i, 0)),
            pl.BlockSpec((BN1, 1), lambda i: (i, 0)),
            pl.BlockSpec((BN1, 1), lambda i: (i, 0)),
        ],
        out_shape=[
            jax.ShapeDtypeStruct((NPAD, 16), _f32),
            jax.ShapeDtypeStruct((NPAD, 1), _f32),
            jax.ShapeDtypeStruct((NPAD, 1), _f32),
        ],
    )(degp, xp)


def _tc2_body(s1p, b_in, a_in, w1, b1, msg):
    s1 = s1p[0, 0] + s1p[1, 0]
    z = (
        jnp.dot(s1 * b_in[...], w1[...], preferred_element_type=_f32)
        + b1[...]
    )
    m = jnp.maximum(z, 0.0) * a_in[...]
    for q in range(4):
        msg[q] = m[:, q * 16 : (q + 1) * 16]


def _tc2(s1p, b_arr, a_arr, w1p, b1r):
    return pl.pallas_call(
        _tc2_body,
        grid=(NPAD // BN2,),
        in_specs=[
            pl.BlockSpec((NC, 1, BN2, 16), lambda i: (0, 0, i, 0)),
            pl.BlockSpec((BN2, 1), lambda i: (i, 0)),
            pl.BlockSpec((BN2, 1), lambda i: (i, 0)),
            pl.BlockSpec((16, H), lambda i: (0, 0)),
            pl.BlockSpec((1, H), lambda i: (0, 0)),
        ],
        out_specs=pl.BlockSpec((4, BN2, 16), lambda i: (0, i, 0)),
        out_shape=jax.ShapeDtypeStruct((4, NPAD, 16), _f32),
    )(s1p, b_arr, a_arr, w1p, b1r)


def _tc3_body(fg, s2p, b_in, gid, w2, b2, wc, bc, loss, hsum, cnt):
    i = pl.program_id(0)

    @pl.when(i == 0)
    def _():
        hsum[...] = jnp.zeros_like(hsum)
        cnt[...] = jnp.zeros_like(cnt)

    sp = s2p[0] + s2p[1]
    s2 = jnp.concatenate([sp[q] for q in range(4)], axis=1)
    z = (
        jnp.dot(s2 * b_in[...], w2[...], preferred_element_type=_f32)
        + b2[...]
    )
    h2 = jnp.maximum(z, 0.0)
    oh = (gid[...] == lax.broadcasted_iota(_i32, (1, G), 1)).astype(_f32)
    hsum[...] += lax.dot_general(
        oh, h2, (((0,), (0,)), ((), ())), preferred_element_type=_f32
    )
    cnt[...] += jnp.sum(oh, axis=0, keepdims=True)

    @pl.when(i == NPAD // BN2 - 1)
    def _():
        hg = hsum[...] / jnp.maximum(cnt[...], 1.0).T
        sc = jnp.dot(hg, wc[...], preferred_element_type=_f32) + bc[...]
        svec = sc[:, 0:1]
        gen = jnp.mean((svec - 1.0) ** 2) * 0.5
        disc = jnp.mean(svec**2) * 0.5
        loss[...] = jnp.where(fg[0, 0] != 0, gen, disc).reshape(1, 1)


def _tc3(fg, s2, b_arr, gid, w2, b2r, wcp, bcp):
    return pl.pallas_call(
        _tc3_body,
        grid=(NPAD // BN2,),
        in_specs=[
            pl.BlockSpec((1, 1), lambda i: (0, 0)),
            pl.BlockSpec((NC, 4, BN2, 16), lambda i: (0, 0, i, 0)),
            pl.BlockSpec((BN2, 1), lambda i: (i, 0)),
            pl.BlockSpec((BN2, 1), lambda i: (i, 0)),
            pl.BlockSpec((H, H), lambda i: (0, 0)),
            pl.BlockSpec((1, H), lambda i: (0, 0)),
            pl.BlockSpec((H, 8), lambda i: (0, 0)),
            pl.BlockSpec((1, 8), lambda i: (0, 0)),
        ],
        out_specs=pl.BlockSpec((1, 1), lambda i: (0, 0)),
        out_shape=jax.ShapeDtypeStruct((1, 1), _f32),
        scratch_shapes=[
            pltpu.VMEM((G, H), _f32),
            pltpu.VMEM((1, G), _f32),
        ],
    )(fg, s2, b_arr, gid, w2, b2r, wcp, bcp)


def kernel(for_gen, x, edge_index, graph_ids, W1, b1, W2, b2, Wc, bc):
    ei_pad = jnp.pad(
        edge_index, ((0, 0), (0, EPAD - E)), constant_values=NPAD - 1
    )
    ei_flat = ei_pad.reshape(2 * EPAD)

    degp = _deg_call(ei_flat)

    xp = jnp.zeros((NPAD, 8), _f32).at[:N, :IN_DIM].set(x)
    x16, a_arr, b_arr = _tc1(degp, xp)

    z16 = jnp.zeros((NPAD, 16), _f32)
    s1p = _l1_call(ei_flat, x16.reshape(1, NPAD, 16), z16)

    w1p = jnp.zeros((16, H), _f32).at[:IN_DIM].set(W1)
    msg = _tc2(s1p, b_arr, a_arr, w1p, b1.reshape(1, H))

    s2p = _l2_call(ei_flat, msg, z16)

    gid = jnp.pad(graph_ids, (0, NPAD - N), constant_values=G).reshape(
        NPAD, 1
    )
    wcp = jnp.zeros((H, 8), _f32).at[:, :1].set(Wc)
    bcp = jnp.zeros((1, 8), _f32).at[0, 0].set(bc[0])
    fg = jnp.asarray(for_gen, _i32).reshape(1, 1)
    loss = _tc3(fg, s2p, b_arr, gid, W2, b2.reshape(1, H), wcp, bcp)
    return loss[0, 0]


# speedup vs baseline: 10.3011x; 1.0410x over previous
import functools

import jax
import jax.numpy as jnp
from jax import lax
from jax.experimental import pallas as pl
from jax.experimental.pallas import tpu as pltpu
from jax.experimental.pallas import tpu_sc as plsc

N = 100000
E = 6400000
G = 64
IN_DIM = 3
H = 64

NC, NS, NW = 2, 16, 32
NPAD = 100352
BN1 = 3584
BN2 = 2048
Q = 200704
EPAD = NW * Q
CHUNK = 2048
NCHUNK = Q // CHUNK
SUBK = 128
NSUB = CHUNK // SUBK
DEG_CHUNK = 4096
DEG_NCHUNK = (2 * Q) // DEG_CHUNK

_f32 = jnp.float32
_i32 = jnp.int32

_MESH = plsc.VectorSubcoreMesh(core_axis_name="c", subcore_axis_name="s")
_SC_PARAMS = pltpu.CompilerParams(
    needs_layout_passes=False, use_tc_tiling_on_sc=False
)


def _deg_body(ei, out, hist, ibufs, sem):
    c = lax.axis_index("c")
    s = lax.axis_index("s")
    zv = jnp.zeros((16,), _f32)

    def zero(i, carry):
        hist[pl.ds(i * 16, 16)] = zv
        return carry

    lax.fori_loop(0, NPAD // 16, zero, 0)

    ones = jnp.ones((16,), _f32)
    base = c * EPAD + s * (2 * Q)
    nvec = DEG_CHUNK // 16

    def scan(buf):
        def inner(i, carry2):
            plsc.addupdate_scatter(hist, [buf[pl.ds(i * 16, 16)]], ones)
            return carry2

        lax.fori_loop(0, nvec, inner, 0)

    def fire(j, buf):
        jc = jnp.where(j < DEG_NCHUNK, j, 0)
        return pltpu.async_copy(
            ei.at[pl.ds(base + jc * DEG_CHUNK, DEG_CHUNK)], buf, sem
        )

    def drain(buf):
        pltpu.make_async_copy(ei.at[pl.ds(base, DEG_CHUNK)], buf, sem).wait()

    fire(0, ibufs[0])

    def chunk2(j2, carry):
        j0 = 2 * j2
        drain(ibufs[0])
        fire(j0 + 1, ibufs[1])
        scan(ibufs[0])
        drain(ibufs[1])
        fire(j0 + 2, ibufs[0])
        scan(ibufs[1])
        return carry

    lax.fori_loop(0, DEG_NCHUNK // 2, chunk2, 0)
    drain(ibufs[0])
    pltpu.sync_copy(hist, out.at[c, s])


_deg_call = functools.partial(
    pl.kernel,
    out_type=jax.ShapeDtypeStruct((NC, NS, NPAD), _f32),
    mesh=_MESH,
    compiler_params=_SC_PARAMS,
    scratch_types=[
        pltpu.VMEM((NPAD,), _f32),
        [pltpu.VMEM((DEG_CHUNK,), _i32) for _ in range(2)],
        pltpu.SemaphoreType.DMA,
    ],
)(_deg_body)


def _make_seg_body(ng):
    def body(ei, table, z16, out, acc, sbufs, dbufs2, rows2, isems, gsem):
        c = lax.axis_index("c")
        s = lax.axis_index("s")
        t = c * NS + s
        base = t * Q
        rows_per_tile = NPAD // NS

        def stage(j, p):
            jc = jnp.where(j < NCHUNK, j, 0)
            off = base + jc * CHUNK
            pltpu.async_copy(ei.at[pl.ds(off, CHUNK)], sbufs[p], isems[p])
            for k in range(NSUB):
                pltpu.async_copy(
                    ei.at[pl.ds(EPAD + off + k * SUBK, SUBK)],
                    dbufs2[p][k],
                    isems[p],
                )

        def drain(p):
            pltpu.make_async_copy(
                ei.at[pl.ds(base, CHUNK)], sbufs[p], isems[p]
            ).wait()
            for k in range(NSUB):
                pltpu.make_async_copy(
                    ei.at[pl.ds(base, SUBK)], dbufs2[p][k], isems[p]
                ).wait()

        for q in range(ng):
            pltpu.sync_copy(
                z16.at[pl.ds(s * rows_per_tile, rows_per_tile)],
                acc.at[pl.ds(s * rows_per_tile, rows_per_tile)],
            )
            plsc.subcore_barrier()

            def work(p):
                sbuf, dbufs = sbufs[p], dbufs2[p]
                g = pltpu.async_copy(
                    table.at[q].at[sbuf.at[pl.ds(0, SUBK)]], rows2[0], gsem[0]
                )
                for k in range(NSUB):
                    g.wait()
                    if k + 1 < NSUB:
                        g = pltpu.async_copy(
                            table.at[q].at[
                                sbuf.at[pl.ds((k + 1) * SUBK, SUBK)]
                            ],
                            rows2[(k + 1) % 2],
                            gsem[(k + 1) % 2],
                        )
                    pltpu.sync_copy(rows2[k % 2], acc.at[dbufs[k]], add=True)

            stage(0, 0)

            def chunk2(j2, carry):
                j0 = 2 * j2
                drain(0)
                stage(j0 + 1, 1)
                work(0)
                drain(1)
                stage(j0 + 2, 0)
                work(1)
                return carry

            lax.fori_loop(0, NCHUNK // 2, chunk2, 0)
            drain(0)
            plsc.subcore_barrier()
            pltpu.sync_copy(
                acc.at[pl.ds(s * rows_per_tile, rows_per_tile)],
                out.at[c, q, pl.ds(s * rows_per_tile, rows_per_tile)],
            )
            plsc.subcore_barrier()

    return body


def _make_seg_call(ng):
    return functools.partial(
        pl.kernel,
        out_type=jax.ShapeDtypeStruct((NC, ng, NPAD, 16), _f32),
        mesh=_MESH,
        compiler_params=_SC_PARAMS,
        scratch_types=[
            pltpu.VMEM_SHARED((NPAD, 16), _f32),
            [pltpu.VMEM((CHUNK,), _i32) for _ in range(2)],
            [
                [pltpu.VMEM((SUBK,), _i32) for _ in range(NSUB)]
                for _ in range(2)
            ],
            [pltpu.VMEM((SUBK, 16), _f32) for _ in range(2)],
            [pltpu.SemaphoreType.DMA for _ in range(2)],
            [pltpu.SemaphoreType.DMA for _ in range(2)],
        ],
    )(_make_seg_body(ng))


_l1_call = _make_seg_call(1)
_l2_call = _make_seg_call(4)


def _tc1_body(degp, xp, x16, a_out, b_out):
    d = jnp.sum(degp[...], axis=1)
    a = lax.rsqrt(jnp.maximum(d[0], 1.0))
    b = lax.rsqrt(jnp.maximum(d[1], 1.0))
    a_out[...] = a[:, None]
    b_out[...] = b[:, None]
    xb = xp[...] * a[:, None]
    x16[...] = jnp.concatenate([xb, jnp.zeros_like(xb)], axis=1)


def _tc1(degp, xp):
    return pl.pallas_call(
        _tc1_body,
        grid=(NPAD // BN1,),
        in_specs=[
            pl.BlockSpec((NC, NS, BN1), lambda i: (0, 0, i)),
            pl.BlockSpec((BN1, 8), lambda i: (i, 0)),
        ],
        out_specs=[
            pl.BlockSpec((BN1, 16), lambda i: (i, 0)),
            pl.BlockSpec((BN1, 1), lambda i: (i, 0)),
            pl.BlockSpec((BN1, 1), lambda i: (i, 0)),
        ],
        out_shape=[
            jax.ShapeDtypeStruct((NPAD, 16), _f32),
            jax.ShapeDtypeStruct((NPAD, 1), _f32),
            jax.ShapeDtypeStruct((NPAD, 1), _f32),
        ],
    )(degp, xp)


def _tc2_body(s1p, b_in, a_in, w1, b1, msg):
    s1 = s1p[0, 0] + s1p[1, 0]
    z = (
        jnp.dot(s1 * b_in[...], w1[...], preferred_element_type=_f32)
        + b1[...]
    )
    m = jnp.maximum(z, 0.0) * a_in[...]
    for q in range(4):
        msg[q] = m[:, q * 16 : (q + 1) * 16]


def _tc2(s1p, b_arr, a_arr, w1p, b1r):
    return pl.pallas_call(
        _tc2_body,
        grid=(NPAD // BN2,),
        in_specs=[
            pl.BlockSpec((NC, 1, BN2, 16), lambda i: (0, 0, i, 0)),
            pl.BlockSpec((BN2, 1), lambda i: (i, 0)),
            pl.BlockSpec((BN2, 1), lambda i: (i, 0)),
            pl.BlockSpec((16, H), lambda i: (0, 0)),
            pl.BlockSpec((1, H), lambda i: (0, 0)),
        ],
        out_specs=pl.BlockSpec((4, BN2, 16), lambda i: (0, i, 0)),
        out_shape=jax.ShapeDtypeStruct((4, NPAD, 16), _f32),
    )(s1p, b_arr, a_arr, w1p, b1r)


def _tc3_body(fg, s2p, b_in, gid, w2, b2, wc, bc, loss, hsum, cnt):
    i = pl.program_id(0)

    @pl.when(i == 0)
    def _():
        hsum[...] = jnp.zeros_like(hsum)
        cnt[...] = jnp.zeros_like(cnt)

    sp = s2p[0] + s2p[1]
    s2 = jnp.concatenate([sp[q] for q in range(4)], axis=1)
    z = (
        jnp.dot(s2 * b_in[...], w2[...], preferred_element_type=_f32)
        + b2[...]
    )
    h2 = jnp.maximum(z, 0.0)
    oh = (gid[...] == lax.broadcasted_iota(_i32, (1, G), 1)).astype(_f32)
    hsum[...] += lax.dot_general(
        oh, h2, (((0,), (0,)), ((), ())), preferred_element_type=_f32
    )
    cnt[...] += jnp.sum(oh, axis=0, keepdims=True)

    @pl.when(i == NPAD // BN2 - 1)
    def _():
        hg = hsum[...] / jnp.maximum(cnt[...], 1.0).T
        sc = jnp.dot(hg, wc[...], preferred_element_type=_f32) + bc[...]
        svec = sc[:, 0:1]
        gen = jnp.mean((svec - 1.0) ** 2) * 0.5
        disc = jnp.mean(svec**2) * 0.5
        loss[...] = jnp.where(fg[0, 0] != 0, gen, disc).reshape(1, 1)


def _tc3(fg, s2, b_arr, gid, w2, b2r, wcp, bcp):
    return pl.pallas_call(
        _tc3_body,
        grid=(NPAD // BN2,),
        in_specs=[
            pl.BlockSpec((1, 1), lambda i: (0, 0)),
            pl.BlockSpec((NC, 4, BN2, 16), lambda i: (0, 0, i, 0)),
            pl.BlockSpec((BN2, 1), lambda i: (i, 0)),
            pl.BlockSpec((BN2, 1), lambda i: (i, 0)),
            pl.BlockSpec((H, H), lambda i: (0, 0)),
            pl.BlockSpec((1, H), lambda i: (0, 0)),
            pl.BlockSpec((H, 8), lambda i: (0, 0)),
            pl.BlockSpec((1, 8), lambda i: (0, 0)),
        ],
        out_specs=pl.BlockSpec((1, 1), lambda i: (0, 0)),
        out_shape=jax.ShapeDtypeStruct((1, 1), _f32),
        scratch_shapes=[
            pltpu.VMEM((G, H), _f32),
            pltpu.VMEM((1, G), _f32),
        ],
    )(fg, s2, b_arr, gid, w2, b2r, wcp, bcp)


def kernel(for_gen, x, edge_index, graph_ids, W1, b1, W2, b2, Wc, bc):
    ei_pad = jnp.pad(
        edge_index, ((0, 0), (0, EPAD - E)), constant_values=NPAD - 1
    )
    ei_flat = ei_pad.reshape(2 * EPAD)

    degp = _deg_call(ei_flat)

    xp = jnp.zeros((NPAD, 8), _f32).at[:N, :IN_DIM].set(x)
    x16, a_arr, b_arr = _tc1(degp, xp)

    z16 = jnp.zeros((NPAD, 16), _f32)
    s1p = _l1_call(ei_flat, x16.reshape(1, NPAD, 16), z16)

    w1p = jnp.zeros((16, H), _f32).at[:IN_DIM].set(W1)
    msg = _tc2(s1p, b_arr, a_arr, w1p, b1.reshape(1, H))

    s2p = _l2_call(ei_flat, msg, z16)

    gid = jnp.pad(graph_ids, (0, NPAD - N), constant_values=G).reshape(
        NPAD, 1
    )
    wcp = jnp.zeros((H, 8), _f32).at[:, :1].set(Wc)
    bcp = jnp.zeros((1, 8), _f32).at[0, 0].set(bc[0])
    fg = jnp.asarray(for_gen, _i32).reshape(1, 1)
    loss = _tc3(fg, s2p, b_arr, gid, W2, b2.reshape(1, H), wcp, bcp)
    return loss[0, 0]
